# baseline (speedup 1.0000x reference)
import functools
import jax
import jax.numpy as jnp
from jax import lax
from jax.experimental import pallas as pl
from jax.experimental.pallas import tpu as pltpu
from jax.experimental.pallas import tpu_sc as plsc

H = 1024
B = 8192
CH = 64

info = plsc.get_sparse_core_info()
NC, NS = info.num_cores, info.num_subcores
NW = NC * NS
B_PER_W = B // NW
NCHUNK = B_PER_W // CH

mesh = plsc.VectorSubcoreMesh(core_axis_name="c", subcore_axis_name="s")


@functools.partial(
    pl.kernel, mesh=mesh,
    out_type=jax.ShapeDtypeStruct((B, H), jnp.float32),
    scratch_types=[
        pltpu.VMEM((CH,), jnp.int32),
        pltpu.VMEM((CH, H), jnp.float32),
        pltpu.SemaphoreType.DMA,
    ],
)
def _sc_gather(table_hbm, idx_hbm, out_hbm, idx_v, rows_v, sem):
    wid = lax.axis_index("s") * NC + lax.axis_index("c")
    for j in range(NCHUNK):
        base = wid * B_PER_W + j * CH
        pltpu.sync_copy(idx_hbm.at[pl.ds(base, CH)], idx_v)
        pltpu.async_copy(table_hbm.at[idx_v], rows_v, sem).wait()
        pltpu.sync_copy(rows_v, out_hbm.at[pl.ds(base, CH)])


def kernel(input, G, gate_proj, up_proj, down_proj, lora_A, lora_B):
    b, s, h = input.shape
    xt = input.reshape(-1, h)
    n = xt.shape[0]
    idx = (jnp.arange(B, dtype=jnp.int32) * 40503 % n).astype(jnp.int32)
    out = _sc_gather(xt, idx)
    return out[:n].reshape(b, s, h)


# speedup vs baseline: 1.1366x; 1.1366x over previous
import functools
import jax
import jax.numpy as jnp
from jax.experimental import pallas as pl
from jax.experimental.pallas import tpu as pltpu

H = 1024
E = 8
FFH = H // E
LORA_R = 16
LORA_SCALE = 2.0
HID = E * FFH
LR = E * LORA_R
TB = 1024


def _fused_kernel(x_ref, g_ref, gp_hbm, up_hbm, la_hbm, dp_hbm, lb_hbm,
                  o_ref, win_s, wout_s, sa, sb, sc, sd, se, sems):
    @pl.when(pl.program_id(0) == 0)
    def _prep():
        cpa = pltpu.make_async_copy(gp_hbm, sa, sems.at[0])
        cpb = pltpu.make_async_copy(up_hbm, sb, sems.at[1])
        cpc = pltpu.make_async_copy(la_hbm, sc, sems.at[2])
        cpd = pltpu.make_async_copy(dp_hbm, sd, sems.at[3])
        cpe = pltpu.make_async_copy(lb_hbm, se, sems.at[4])
        cpa.start(); cpb.start(); cpc.start(); cpd.start(); cpe.start()
        cpa.wait()
        win_s[:, :HID] = sa[...].T.astype(jnp.bfloat16)
        cpb.wait()
        win_s[:, HID:2 * HID] = sb[...].T.astype(jnp.bfloat16)
        cpc.wait()
        win_s[:, 2 * HID:] = sc[...].T.astype(jnp.bfloat16)
        cpd.wait()
        for e in range(E):
            wout_s[e * FFH:(e + 1) * FFH, :] = sd[e].T.astype(jnp.bfloat16)
        cpe.wait()
        for e in range(E):
            wout_s[HID + e * LORA_R:HID + (e + 1) * LORA_R, :] = (
                (se[e].T * LORA_SCALE).astype(jnp.bfloat16))

    xb = x_ref[...]

    logits = jnp.dot(xb, g_ref[...], preferred_element_type=jnp.float32)
    col = jax.lax.broadcasted_iota(jnp.int32, logits.shape, 1)
    logits = jnp.where(col < E, logits, -1e30)
    m1 = jnp.max(logits, axis=-1, keepdims=True)
    idx1 = jnp.min(jnp.where(logits == m1, col, E), axis=-1, keepdims=True)
    l2 = jnp.where(col == idx1, -1e30, logits)
    m2 = jnp.max(l2, axis=-1, keepdims=True)
    idx2 = jnp.min(jnp.where(l2 == m2, col, E), axis=-1, keepdims=True)
    t = jnp.exp(m2 - m1)
    w1 = 1.0 / (1.0 + t)
    w2 = t / (1.0 + t)

    xb16 = xb.astype(jnp.bfloat16)
    a = jnp.dot(xb16, win_s[:, :HID], preferred_element_type=jnp.float32)
    u = jnp.dot(xb16, win_s[:, HID:2 * HID], preferred_element_type=jnp.float32)
    l = jnp.dot(xb16, win_s[:, 2 * HID:], preferred_element_type=jnp.float32)
    h = (a / (1.0 + jnp.exp(-a))) * u

    hcol = jax.lax.broadcasted_iota(jnp.int32, h.shape, 1) // FFH
    wh = jnp.where(hcol == idx1, w1, 0.0) + jnp.where(hcol == idx2, w2, 0.0)
    lcol = jax.lax.broadcasted_iota(jnp.int32, l.shape, 1) // LORA_R
    wl = jnp.where(lcol == idx1, w1, 0.0) + jnp.where(lcol == idx2, w2, 0.0)
    hw = (h * wh).astype(jnp.bfloat16)
    lw = (l * wl).astype(jnp.bfloat16)

    o_ref[...] = (
        jnp.dot(hw, wout_s[:HID, :], preferred_element_type=jnp.float32)
        + jnp.dot(lw, wout_s[HID:, :], preferred_element_type=jnp.float32))


@functools.partial(jax.jit, static_argnames=("interpret",))
def _run(xt, g_pad, gp_r, up_r, la_r, dp, lb, interpret=False):
    n = xt.shape[0]
    anyspec = pl.BlockSpec(memory_space=pl.ANY)
    return pl.pallas_call(
        _fused_kernel,
        grid=(n // TB,),
        in_specs=[
            pl.BlockSpec((TB, H), lambda i: (i, 0)),
            pl.BlockSpec((H, 128), lambda i: (0, 0)),
            anyspec, anyspec, anyspec, anyspec, anyspec,
        ],
        out_specs=pl.BlockSpec((TB, H), lambda i: (i, 0)),
        out_shape=jax.ShapeDtypeStruct((n, H), jnp.float32),
        scratch_shapes=[
            pltpu.VMEM((H, 2 * HID + LR), jnp.bfloat16),
            pltpu.VMEM((HID + LR, H), jnp.bfloat16),
            pltpu.VMEM((HID, H), jnp.float32),
            pltpu.VMEM((HID, H), jnp.float32),
            pltpu.VMEM((LR, H), jnp.float32),
            pltpu.VMEM((E, H, FFH), jnp.float32),
            pltpu.VMEM((E, H, LORA_R), jnp.float32),
            pltpu.SemaphoreType.DMA((5,)),
        ],
        compiler_params=pltpu.CompilerParams(
            dimension_semantics=("arbitrary",)),
        interpret=interpret,
    )(xt, g_pad, gp_r, up_r, la_r, dp, lb)


def kernel(input, G, gate_proj, up_proj, down_proj, lora_A, lora_B,
           interpret=False):
    b, s, h = input.shape
    xt = input.reshape(-1, h)
    g_pad = jnp.pad(G, ((0, 0), (0, 128 - E)))
    gp_r = gate_proj.reshape(HID, H)
    up_r = up_proj.reshape(HID, H)
    la_r = lora_A.reshape(LR, H)
    out = _run(xt, g_pad, gp_r, up_r, la_r, down_proj, lora_B,
               interpret=interpret)
    return out.reshape(b, s, h)


# speedup vs baseline: 1.1479x; 1.0099x over previous
import functools
import jax
import jax.numpy as jnp
from jax.experimental import pallas as pl
from jax.experimental.pallas import tpu as pltpu

H = 1024
E = 8
FFH = H // E
LORA_R = 16
LORA_SCALE = 2.0
HID = E * FFH
LR = E * LORA_R
TB = 512


def _fused_kernel(x_ref, g_ref, gp_hbm, up_hbm, la_hbm, dp_hbm, lb_hbm,
                  o_ref, win_s, wout_s, sa, sb, sc, sd, se, sems):
    @pl.when(pl.program_id(0) == 0)
    def _prep():
        cpa = pltpu.make_async_copy(gp_hbm, sa, sems.at[0])
        cpb = pltpu.make_async_copy(up_hbm, sb, sems.at[1])
        cpc = pltpu.make_async_copy(la_hbm, sc, sems.at[2])
        cpd = pltpu.make_async_copy(dp_hbm, sd, sems.at[3])
        cpe = pltpu.make_async_copy(lb_hbm, se, sems.at[4])
        cpa.start(); cpb.start(); cpc.start(); cpd.start(); cpe.start()
        cpa.wait()
        win_s[:, :HID] = sa[...].T.astype(jnp.bfloat16)
        cpb.wait()
        win_s[:, HID:2 * HID] = sb[...].T.astype(jnp.bfloat16)
        cpc.wait()
        win_s[:, 2 * HID:] = sc[...].T.astype(jnp.bfloat16)
        cpd.wait()
        for e in range(E):
            wout_s[e * FFH:(e + 1) * FFH, :] = sd[e].T.astype(jnp.bfloat16)
        cpe.wait()
        for e in range(E):
            wout_s[HID + e * LORA_R:HID + (e + 1) * LORA_R, :] = (
                (se[e].T * LORA_SCALE).astype(jnp.bfloat16))

    xb = x_ref[...]

    logits = jnp.dot(xb, g_ref[...], preferred_element_type=jnp.float32)
    col = jax.lax.broadcasted_iota(jnp.int32, logits.shape, 1)
    logits = jnp.where(col < E, logits, -1e30)
    m1 = jnp.max(logits, axis=-1, keepdims=True)
    idx1 = jnp.min(jnp.where(logits == m1, col, E), axis=-1, keepdims=True)
    l2 = jnp.where(col == idx1, -1e30, logits)
    m2 = jnp.max(l2, axis=-1, keepdims=True)
    idx2 = jnp.min(jnp.where(l2 == m2, col, E), axis=-1, keepdims=True)
    t = jnp.exp(m2 - m1)
    w1 = 1.0 / (1.0 + t)
    w2 = t / (1.0 + t)

    xb16 = xb.astype(jnp.bfloat16)
    a = jnp.dot(xb16, win_s[:, :HID], preferred_element_type=jnp.float32)
    u = jnp.dot(xb16, win_s[:, HID:2 * HID], preferred_element_type=jnp.float32)
    l = jnp.dot(xb16, win_s[:, 2 * HID:], preferred_element_type=jnp.float32)
    h = (a / (1.0 + jnp.exp(-a))) * u

    hcol = jax.lax.broadcasted_iota(jnp.int32, h.shape, 1) // FFH
    wh = jnp.where(hcol == idx1, w1, 0.0) + jnp.where(hcol == idx2, w2, 0.0)
    lcol = jax.lax.broadcasted_iota(jnp.int32, l.shape, 1) // LORA_R
    wl = jnp.where(lcol == idx1, w1, 0.0) + jnp.where(lcol == idx2, w2, 0.0)
    hw = (h * wh).astype(jnp.bfloat16)
    lw = (l * wl).astype(jnp.bfloat16)

    o_ref[...] = (
        jnp.dot(hw, wout_s[:HID, :], preferred_element_type=jnp.float32)
        + jnp.dot(lw, wout_s[HID:, :], preferred_element_type=jnp.float32))


@functools.partial(jax.jit, static_argnames=("interpret",))
def _run(xt, g_pad, gp_r, up_r, la_r, dp, lb, interpret=False):
    n = xt.shape[0]
    anyspec = pl.BlockSpec(memory_space=pl.ANY)
    return pl.pallas_call(
        _fused_kernel,
        grid=(n // TB,),
        in_specs=[
            pl.BlockSpec((TB, H), lambda i: (i, 0)),
            pl.BlockSpec((H, 128), lambda i: (0, 0)),
            anyspec, anyspec, anyspec, anyspec, anyspec,
        ],
        out_specs=pl.BlockSpec((TB, H), lambda i: (i, 0)),
        out_shape=jax.ShapeDtypeStruct((n, H), jnp.float32),
        scratch_shapes=[
            pltpu.VMEM((H, 2 * HID + LR), jnp.bfloat16),
            pltpu.VMEM((HID + LR, H), jnp.bfloat16),
            pltpu.VMEM((HID, H), jnp.float32),
            pltpu.VMEM((HID, H), jnp.float32),
            pltpu.VMEM((LR, H), jnp.float32),
            pltpu.VMEM((E, H, FFH), jnp.float32),
            pltpu.VMEM((E, H, LORA_R), jnp.float32),
            pltpu.SemaphoreType.DMA((5,)),
        ],
        compiler_params=pltpu.CompilerParams(
            dimension_semantics=("arbitrary",)),
        interpret=interpret,
    )(xt, g_pad, gp_r, up_r, la_r, dp, lb)


def kernel(input, G, gate_proj, up_proj, down_proj, lora_A, lora_B,
           interpret=False):
    b, s, h = input.shape
    xt = input.reshape(-1, h)
    g_pad = jnp.pad(G, ((0, 0), (0, 128 - E)))
    gp_r = gate_proj.reshape(HID, H)
    up_r = up_proj.reshape(HID, H)
    la_r = lora_A.reshape(LR, H)
    out = _run(xt, g_pad, gp_r, up_r, la_r, down_proj, lora_B,
               interpret=interpret)
    return out.reshape(b, s, h)


# speedup vs baseline: 1.1591x; 1.0098x over previous
import functools
import jax
import jax.numpy as jnp
from jax.experimental import pallas as pl
from jax.experimental.pallas import tpu as pltpu

H = 1024
E = 8
FFH = H // E
LORA_R = 16
LORA_SCALE = 2.0
HID = E * FFH
LR = E * LORA_R
TB = 512


def _fused_kernel(x_ref, g_ref, gp_hbm, up_hbm, la_hbm, dp_hbm, lb_hbm,
                  o_ref, win_s, wout_s, sa, sb, sc, sd, se, sems):
    @pl.when(pl.program_id(0) == 0)
    def _prep():
        cpa = pltpu.make_async_copy(gp_hbm, sa, sems.at[0])
        cpb = pltpu.make_async_copy(up_hbm, sb, sems.at[1])
        cpc = pltpu.make_async_copy(la_hbm, sc, sems.at[2])
        cpd = pltpu.make_async_copy(dp_hbm, sd, sems.at[3])
        cpe = pltpu.make_async_copy(lb_hbm, se, sems.at[4])
        cpa.start(); cpb.start(); cpc.start(); cpd.start(); cpe.start()
        cpa.wait()
        win_s[:, :HID] = sa[...].T.astype(jnp.bfloat16)
        cpb.wait()
        win_s[:, HID:2 * HID] = sb[...].T.astype(jnp.bfloat16)
        cpc.wait()
        win_s[:, 2 * HID:] = sc[...].T.astype(jnp.bfloat16)
        cpd.wait()
        for e in range(E):
            wout_s[e * FFH:(e + 1) * FFH, :] = sd[e].T.astype(jnp.bfloat16)
        cpe.wait()
        for e in range(E):
            wout_s[HID + e * LORA_R:HID + (e + 1) * LORA_R, :] = (
                (se[e].T * LORA_SCALE).astype(jnp.bfloat16))

    xb = x_ref[...]

    logits = jnp.dot(xb, g_ref[...], preferred_element_type=jnp.float32)
    col = jax.lax.broadcasted_iota(jnp.int32, logits.shape, 1)
    logits = jnp.where(col < E, logits, -1e30)
    m1 = jnp.max(logits, axis=-1, keepdims=True)
    idx1 = jnp.min(jnp.where(logits == m1, col, E), axis=-1, keepdims=True)
    l2 = jnp.where(col == idx1, -1e30, logits)
    m2 = jnp.max(l2, axis=-1, keepdims=True)
    idx2 = jnp.min(jnp.where(l2 == m2, col, E), axis=-1, keepdims=True)
    t = jnp.exp(m2 - m1)
    w1 = 1.0 / (1.0 + t)
    w2 = t / (1.0 + t)

    xb16 = xb.astype(jnp.bfloat16)
    HH = HID // 2

    def half(lo):
        a = jnp.dot(xb16, win_s[:, lo:lo + HH],
                    preferred_element_type=jnp.float32)
        u = jnp.dot(xb16, win_s[:, HID + lo:HID + lo + HH],
                    preferred_element_type=jnp.float32)
        h = (a / (1.0 + jnp.exp(-a))) * u
        hcol = (jax.lax.broadcasted_iota(jnp.int32, h.shape, 1) + lo) // FFH
        wh = (jnp.where(hcol == idx1, w1, 0.0)
              + jnp.where(hcol == idx2, w2, 0.0))
        return (h * wh).astype(jnp.bfloat16)

    hw0 = half(0)
    hw1 = half(HH)
    l = jnp.dot(xb16, win_s[:, 2 * HID:], preferred_element_type=jnp.float32)
    lcol = jax.lax.broadcasted_iota(jnp.int32, l.shape, 1) // LORA_R
    wl = jnp.where(lcol == idx1, w1, 0.0) + jnp.where(lcol == idx2, w2, 0.0)
    lw = (l * wl).astype(jnp.bfloat16)

    o_ref[...] = (
        jnp.dot(hw0, wout_s[:HH, :], preferred_element_type=jnp.float32)
        + jnp.dot(hw1, wout_s[HH:HID, :], preferred_element_type=jnp.float32)
        + jnp.dot(lw, wout_s[HID:, :], preferred_element_type=jnp.float32))


@functools.partial(jax.jit, static_argnames=("interpret",))
def _run(xt, g_pad, gp_r, up_r, la_r, dp, lb, interpret=False):
    n = xt.shape[0]
    anyspec = pl.BlockSpec(memory_space=pl.ANY)
    return pl.pallas_call(
        _fused_kernel,
        grid=(n // TB,),
        in_specs=[
            pl.BlockSpec((TB, H), lambda i: (i, 0)),
            pl.BlockSpec((H, 128), lambda i: (0, 0)),
            anyspec, anyspec, anyspec, anyspec, anyspec,
        ],
        out_specs=pl.BlockSpec((TB, H), lambda i: (i, 0)),
        out_shape=jax.ShapeDtypeStruct((n, H), jnp.float32),
        scratch_shapes=[
            pltpu.VMEM((H, 2 * HID + LR), jnp.bfloat16),
            pltpu.VMEM((HID + LR, H), jnp.bfloat16),
            pltpu.VMEM((HID, H), jnp.float32),
            pltpu.VMEM((HID, H), jnp.float32),
            pltpu.VMEM((LR, H), jnp.float32),
            pltpu.VMEM((E, H, FFH), jnp.float32),
            pltpu.VMEM((E, H, LORA_R), jnp.float32),
            pltpu.SemaphoreType.DMA((5,)),
        ],
        compiler_params=pltpu.CompilerParams(
            dimension_semantics=("arbitrary",)),
        interpret=interpret,
    )(xt, g_pad, gp_r, up_r, la_r, dp, lb)


def kernel(input, G, gate_proj, up_proj, down_proj, lora_A, lora_B,
           interpret=False):
    b, s, h = input.shape
    xt = input.reshape(-1, h)
    g_pad = jnp.pad(G, ((0, 0), (0, 128 - E)))
    gp_r = gate_proj.reshape(HID, H)
    up_r = up_proj.reshape(HID, H)
    la_r = lora_A.reshape(LR, H)
    out = _run(xt, g_pad, gp_r, up_r, la_r, down_proj, lora_B,
               interpret=interpret)
    return out.reshape(b, s, h)
